# baseline (speedup 1.0000x reference)
import jax
import jax.numpy as jnp
from jax.experimental import pallas as pl
from jax.experimental.pallas import tpu as pltpu

_B, _H, _S, _D, _Q = 8, 16, 4096, 128, 8
_BH = _B * _H
_CHUNK = 8
_W = 8


def _kv_kernel(pos_ref, kc_ref, vc_ref, kv_ref, vv_ref, ko_ref, vo_ref,
               kwin_ref, vwin_ref, copy_sem, row_sem):
    copies = []
    for i in range(0, _BH, _CHUNK):
        copies.append(pltpu.make_async_copy(
            kc_ref.at[pl.ds(i, _CHUNK)], ko_ref.at[pl.ds(i, _CHUNK)], copy_sem))
        copies.append(pltpu.make_async_copy(
            vc_ref.at[pl.ds(i, _CHUNK)], vo_ref.at[pl.ds(i, _CHUNK)], copy_sem))
    for c in copies:
        c.start()
    for c in copies:
        c.wait()
    sub_iota = jax.lax.broadcasted_iota(jnp.int32, (1, _W, 1), 1)
    for q in range(_Q):
        pos = pos_ref[q]
        wb = pl.multiple_of((pos // _W) * _W, _W)
        r = pos % _W
        cki = pltpu.make_async_copy(ko_ref.at[:, pl.ds(wb, _W), :], kwin_ref, row_sem)
        cvi = pltpu.make_async_copy(vo_ref.at[:, pl.ds(wb, _W), :], vwin_ref, row_sem)
        cki.start()
        cvi.start()
        cki.wait()
        cvi.wait()
        mask = sub_iota == r
        kwin_ref[...] = jnp.where(mask, kv_ref[:, q:q + 1, :], kwin_ref[...])
        vwin_ref[...] = jnp.where(mask, vv_ref[:, q:q + 1, :], vwin_ref[...])
        cko = pltpu.make_async_copy(kwin_ref, ko_ref.at[:, pl.ds(wb, _W), :], row_sem)
        cvo = pltpu.make_async_copy(vwin_ref, vo_ref.at[:, pl.ds(wb, _W), :], row_sem)
        cko.start()
        cvo.start()
        cko.wait()
        cvo.wait()


def kernel(k_cache, v_cache, input_pos, k_val, v_val):
    kc = k_cache.reshape(_BH, _S, _D)
    vc = v_cache.reshape(_BH, _S, _D)
    kv = k_val.reshape(_BH, _Q, _D)
    vv = v_val.reshape(_BH, _Q, _D)
    ko, vo = pl.pallas_call(
        _kv_kernel,
        in_specs=[
            pl.BlockSpec(memory_space=pltpu.SMEM),
            pl.BlockSpec(memory_space=pltpu.MemorySpace.HBM),
            pl.BlockSpec(memory_space=pltpu.MemorySpace.HBM),
            pl.BlockSpec(memory_space=pltpu.MemorySpace.VMEM),
            pl.BlockSpec(memory_space=pltpu.MemorySpace.VMEM),
        ],
        out_specs=(pl.BlockSpec(memory_space=pltpu.MemorySpace.HBM),
                   pl.BlockSpec(memory_space=pltpu.MemorySpace.HBM)),
        out_shape=(jax.ShapeDtypeStruct((_BH, _S, _D), kc.dtype),
                   jax.ShapeDtypeStruct((_BH, _S, _D), vc.dtype)),
        scratch_shapes=[
            pltpu.VMEM((_BH, _W, _D), jnp.bfloat16),
            pltpu.VMEM((_BH, _W, _D), jnp.bfloat16),
            pltpu.SemaphoreType.DMA,
            pltpu.SemaphoreType.DMA,
        ],
    )(input_pos, kc, vc, kv, vv)
    K = ko.reshape(_B, _H, _S, _D)
    V = vo.reshape(_B, _H, _S, _D)
    return (K, K, V)


# speedup vs baseline: 30.7570x; 30.7570x over previous
import jax
import jax.numpy as jnp
from jax.experimental import pallas as pl
from jax.experimental.pallas import tpu as pltpu

_B, _H, _S, _D, _Q = 8, 16, 4096, 128, 8
_BH = _B * _H
_CHUNK = 8
_W = 8


def _kv_kernel(pos_ref, kc_ref, vc_ref, kv_ref, vv_ref, ko_ref, vo_ref,
               kwin_ref, vwin_ref, copy_sem, row_sem):
    del kc_ref, vc_ref, copy_sem
    sub_iota = jax.lax.broadcasted_iota(jnp.int32, (1, _W, 1), 1)
    for q in range(_Q):
        pos = pos_ref[q]
        wb = pl.multiple_of((pos // _W) * _W, _W)
        r = pos % _W
        cki = pltpu.make_async_copy(ko_ref.at[:, pl.ds(wb, _W), :], kwin_ref, row_sem)
        cvi = pltpu.make_async_copy(vo_ref.at[:, pl.ds(wb, _W), :], vwin_ref, row_sem)
        cki.start()
        cvi.start()
        cki.wait()
        cvi.wait()
        mask = sub_iota == r
        kwin_ref[...] = jnp.where(mask, kv_ref[:, q:q + 1, :], kwin_ref[...])
        vwin_ref[...] = jnp.where(mask, vv_ref[:, q:q + 1, :], vwin_ref[...])
        cko = pltpu.make_async_copy(kwin_ref, ko_ref.at[:, pl.ds(wb, _W), :], row_sem)
        cvo = pltpu.make_async_copy(vwin_ref, vo_ref.at[:, pl.ds(wb, _W), :], row_sem)
        cko.start()
        cvo.start()
        cko.wait()
        cvo.wait()


def kernel(k_cache, v_cache, input_pos, k_val, v_val):
    kc = k_cache.reshape(_BH, _S, _D)
    vc = v_cache.reshape(_BH, _S, _D)
    kv = k_val.reshape(_BH, _Q, _D)
    vv = v_val.reshape(_BH, _Q, _D)
    ko, vo = pl.pallas_call(
        _kv_kernel,
        in_specs=[
            pl.BlockSpec(memory_space=pltpu.SMEM),
            pl.BlockSpec(memory_space=pltpu.MemorySpace.HBM),
            pl.BlockSpec(memory_space=pltpu.MemorySpace.HBM),
            pl.BlockSpec(memory_space=pltpu.MemorySpace.VMEM),
            pl.BlockSpec(memory_space=pltpu.MemorySpace.VMEM),
        ],
        out_specs=(pl.BlockSpec(memory_space=pltpu.MemorySpace.HBM),
                   pl.BlockSpec(memory_space=pltpu.MemorySpace.HBM)),
        out_shape=(jax.ShapeDtypeStruct((_BH, _S, _D), kc.dtype),
                   jax.ShapeDtypeStruct((_BH, _S, _D), vc.dtype)),
        scratch_shapes=[
            pltpu.VMEM((_BH, _W, _D), jnp.bfloat16),
            pltpu.VMEM((_BH, _W, _D), jnp.bfloat16),
            pltpu.SemaphoreType.DMA,
            pltpu.SemaphoreType.DMA,
        ],
        input_output_aliases={1: 0, 2: 1},
    )(input_pos, kc, vc, kv, vv)
    K = ko.reshape(_B, _H, _S, _D)
    V = vo.reshape(_B, _H, _S, _D)
    return (K, K, V)


# speedup vs baseline: 32.8141x; 1.0669x over previous
import jax
import jax.numpy as jnp
from jax.experimental import pallas as pl
from jax.experimental.pallas import tpu as pltpu

_B, _H, _S, _D, _Q = 8, 16, 4096, 128, 8
_BH = _B * _H
_BHB = 16
_BS = 1024
_W = 8


def _fused_kernel(pos_ref, kc_ref, vc_ref, kv_ref, vv_ref, ko_ref, vo_ref):
    j = pl.program_id(1)
    ko_ref[...] = kc_ref[...]
    vo_ref[...] = vc_ref[...]
    s0 = j * _BS
    sub_iota = jax.lax.broadcasted_iota(jnp.int32, (1, _W, 1), 1)
    for q in range(_Q):
        local = pos_ref[q] - s0
        @pl.when(jnp.logical_and(local >= 0, local < _BS))
        def _():
            wb = pl.multiple_of((local // _W) * _W, _W)
            r = local % _W
            mask = sub_iota == r
            ko_ref[:, pl.ds(wb, _W), :] = jnp.where(
                mask, kv_ref[:, q:q + 1, :], ko_ref[:, pl.ds(wb, _W), :])
            vo_ref[:, pl.ds(wb, _W), :] = jnp.where(
                mask, vv_ref[:, q:q + 1, :], vo_ref[:, pl.ds(wb, _W), :])


def kernel(k_cache, v_cache, input_pos, k_val, v_val):
    kc = k_cache.reshape(_BH, _S, _D)
    vc = v_cache.reshape(_BH, _S, _D)
    kv = k_val.reshape(_BH, _Q, _D)
    vv = v_val.reshape(_BH, _Q, _D)
    grid = (_BH // _BHB, _S // _BS)
    cache_spec = pl.BlockSpec((_BHB, _BS, _D), lambda i, j: (i, j, 0))
    val_spec = pl.BlockSpec((_BHB, _Q, _D), lambda i, j: (i, 0, 0))
    ko, vo = pl.pallas_call(
        _fused_kernel,
        grid=grid,
        in_specs=[
            pl.BlockSpec(memory_space=pltpu.SMEM),
            cache_spec, cache_spec, val_spec, val_spec,
        ],
        out_specs=(cache_spec, cache_spec),
        out_shape=(jax.ShapeDtypeStruct((_BH, _S, _D), kc.dtype),
                   jax.ShapeDtypeStruct((_BH, _S, _D), vc.dtype)),
    )(input_pos, kc, vc, kv, vv)
    K = ko.reshape(_B, _H, _S, _D)
    V = vo.reshape(_B, _H, _S, _D)
    return (K, K, V)


# speedup vs baseline: 32.8715x; 1.0018x over previous
import jax
import jax.numpy as jnp
from jax.experimental import pallas as pl
from jax.experimental.pallas import tpu as pltpu

_B, _H, _S, _D, _Q = 8, 16, 4096, 128, 8
_BH = _B * _H
_BHB = 4
_W = 8


def _fused_kernel(pos_ref, kc_ref, vc_ref, kv_ref, vv_ref, ko_ref, vo_ref):
    ko_ref[...] = kc_ref[...]
    vo_ref[...] = vc_ref[...]
    sub_iota = jax.lax.broadcasted_iota(jnp.int32, (1, _W, 1), 1)
    for q in range(_Q):
        pos = pos_ref[q]
        wb = pl.multiple_of((pos // _W) * _W, _W)
        r = pos % _W
        mask = sub_iota == r
        ko_ref[:, pl.ds(wb, _W), :] = jnp.where(
            mask, kv_ref[:, q:q + 1, :], ko_ref[:, pl.ds(wb, _W), :])
        vo_ref[:, pl.ds(wb, _W), :] = jnp.where(
            mask, vv_ref[:, q:q + 1, :], vo_ref[:, pl.ds(wb, _W), :])


def kernel(k_cache, v_cache, input_pos, k_val, v_val):
    kc = k_cache.reshape(_BH, _S, _D)
    vc = v_cache.reshape(_BH, _S, _D)
    kv = k_val.reshape(_BH, _Q, _D)
    vv = v_val.reshape(_BH, _Q, _D)
    grid = (_BH // _BHB,)
    cache_spec = pl.BlockSpec((_BHB, _S, _D), lambda i: (i, 0, 0))
    val_spec = pl.BlockSpec((_BHB, _Q, _D), lambda i: (i, 0, 0))
    ko, vo = pl.pallas_call(
        _fused_kernel,
        grid=grid,
        in_specs=[
            pl.BlockSpec(memory_space=pltpu.SMEM),
            cache_spec, cache_spec, val_spec, val_spec,
        ],
        out_specs=(cache_spec, cache_spec),
        out_shape=(jax.ShapeDtypeStruct((_BH, _S, _D), kc.dtype),
                   jax.ShapeDtypeStruct((_BH, _S, _D), vc.dtype)),
    )(input_pos, kc, vc, kv, vv)
    K = ko.reshape(_B, _H, _S, _D)
    V = vo.reshape(_B, _H, _S, _D)
    return (K, K, V)
